# Initial kernel scaffold; baseline (speedup 1.0000x reference)
import jax
import jax.numpy as jnp
from jax.experimental import pallas as pl


def kernel(feat, pool_w, pool_b, fc1_w, fc1_b, fc2_w, fc2_b):
    raise NotImplementedError("write your pallas kernel here")


# speedup vs baseline: 1.4699x; 1.4699x over previous
import functools

import jax
import jax.numpy as jnp
from jax.experimental import pallas as pl
from jax.experimental.pallas import tpu as pltpu

_TB = 8


def _conv_kernel(x_ref, w_ref, b_ref, o_ref):
    w = w_ref[...].astype(jnp.bfloat16)
    b = b_ref[...]
    for t in range(_TB):
        x = x_ref[t].astype(jnp.bfloat16)
        acc = jnp.dot(w, x, preferred_element_type=jnp.float32)
        o_ref[t] = (acc + b).astype(jnp.bfloat16)


def _linear_kernel(x_ref, w_ref, b_ref, o_ref, *, relu):
    w = w_ref[...].astype(jnp.bfloat16)
    acc = jnp.dot(x_ref[...], w, preferred_element_type=jnp.float32)
    acc = acc + b_ref[...]
    if relu:
        acc = jnp.maximum(acc, 0.0)
    o_ref[...] = acc.astype(o_ref.dtype)


def _linear(x, w, b, *, relu, tn, out_dtype):
    B, K = x.shape
    N = w.shape[1]
    bias = b.reshape(1, N)
    return pl.pallas_call(
        functools.partial(_linear_kernel, relu=relu),
        out_shape=jax.ShapeDtypeStruct((B, N), out_dtype),
        grid=(pl.cdiv(N, tn),),
        in_specs=[
            pl.BlockSpec((B, K), lambda j: (0, 0)),
            pl.BlockSpec((K, tn), lambda j: (0, j)),
            pl.BlockSpec((1, tn), lambda j: (0, j)),
        ],
        out_specs=pl.BlockSpec((B, tn), lambda j: (0, j)),
        compiler_params=pltpu.CompilerParams(
            dimension_semantics=("parallel",)),
    )(x, w, bias)


@jax.jit
def _forward(feat, pool_w, pool_b, fc1_w, fc1_b, fc2_w, fc2_b):
    B, C, H, W = feat.shape
    C_out = pool_w.shape[0]
    HW = H * W

    x = feat.reshape(B, C, HW)
    bias = pool_b.reshape(C_out, 1)

    conv_out = pl.pallas_call(
        _conv_kernel,
        out_shape=jax.ShapeDtypeStruct((B, C_out, HW), jnp.bfloat16),
        grid=(B // _TB,),
        in_specs=[
            pl.BlockSpec((_TB, C, HW), lambda i: (i, 0, 0)),
            pl.BlockSpec((C_out, C), lambda i: (0, 0)),
            pl.BlockSpec((C_out, 1), lambda i: (0, 0)),
        ],
        out_specs=pl.BlockSpec((_TB, C_out, HW), lambda i: (i, 0, 0)),
        compiler_params=pltpu.CompilerParams(
            dimension_semantics=("parallel",)),
    )(x, pool_w, bias)

    xf = conv_out.reshape(B, C_out * HW)
    h = _linear(xf, fc1_w, fc1_b, relu=True, tn=512,
                out_dtype=jnp.bfloat16)
    y = _linear(h, fc2_w, fc2_b, relu=False, tn=256,
                out_dtype=jnp.float32)
    return y.reshape(B, 37, 10, 4)


def kernel(feat, pool_w, pool_b, fc1_w, fc1_b, fc2_w, fc2_b):
    return _forward(feat, pool_w, pool_b, fc1_w, fc1_b, fc2_w, fc2_b)


# speedup vs baseline: 1.8991x; 1.2920x over previous
import jax
import jax.numpy as jnp
from jax.experimental import pallas as pl
from jax.experimental.pallas import tpu as pltpu

_THW = 25


def _fused_kernel(x_ref, pw_ref, pb_ref, w1_ref, b1_ref, w2_ref, b2_ref,
                  o_ref, h_ref):
    i = pl.program_id(0)
    thw, b, c = x_ref.shape

    pw = pw_ref[...].astype(jnp.bfloat16)
    x = x_ref[...].astype(jnp.bfloat16).reshape(thw * b, c)
    p2 = jnp.dot(x, pw, preferred_element_type=jnp.float32)
    p2 = (p2 + pb_ref[...]).astype(jnp.bfloat16)

    acc = None
    for j in range(thw):
        pj = p2[j * b:(j + 1) * b, :]
        w1j = w1_ref[:, i * thw + j, :].astype(jnp.bfloat16)
        d = jnp.dot(pj, w1j, preferred_element_type=jnp.float32)
        acc = d if acc is None else acc + d

    @pl.when(i == 0)
    def _():
        h_ref[...] = acc

    @pl.when(i > 0)
    def _():
        h_ref[...] += acc

    @pl.when(i == pl.num_programs(0) - 1)
    def _():
        hr = jnp.maximum(h_ref[...] + b1_ref[...], 0.0).astype(jnp.bfloat16)
        w2 = w2_ref[...].astype(jnp.bfloat16)
        y = jnp.dot(hr, w2, preferred_element_type=jnp.float32)
        o_ref[...] = y + b2_ref[...]


@jax.jit
def _forward(feat, pool_w, pool_b, fc1_w, fc1_b, fc2_w, fc2_b):
    B, C, H, W = feat.shape
    HW = H * W
    N1 = fc1_w.shape[1]
    N2 = fc2_w.shape[1]

    xt = feat.reshape(B, C, HW).transpose(2, 0, 1)
    w1v = fc1_w.reshape(8, HW, N1)

    y = pl.pallas_call(
        _fused_kernel,
        out_shape=jax.ShapeDtypeStruct((B, N2), jnp.float32),
        grid=(HW // _THW,),
        in_specs=[
            pl.BlockSpec((_THW, B, C), lambda i: (i, 0, 0)),
            pl.BlockSpec((C, 8), lambda i: (0, 0)),
            pl.BlockSpec((1, 8), lambda i: (0, 0)),
            pl.BlockSpec((8, HW, N1), lambda i: (0, 0, 0)),
            pl.BlockSpec((1, N1), lambda i: (0, 0)),
            pl.BlockSpec((N1, N2), lambda i: (0, 0)),
            pl.BlockSpec((1, N2), lambda i: (0, 0)),
        ],
        out_specs=pl.BlockSpec((B, N2), lambda i: (0, 0)),
        scratch_shapes=[pltpu.VMEM((B, N1), jnp.float32)],
        compiler_params=pltpu.CompilerParams(
            dimension_semantics=("arbitrary",)),
    )(xt, pool_w.T, pool_b.reshape(1, 8), w1v, fc1_b.reshape(1, N1),
      fc2_w, fc2_b.reshape(1, N2))
    return y.reshape(B, 37, 10, 4)


def kernel(feat, pool_w, pool_b, fc1_w, fc1_b, fc2_w, fc2_b):
    return _forward(feat, pool_w, pool_b, fc1_w, fc1_b, fc2_w, fc2_b)


# speedup vs baseline: 2.1429x; 1.1283x over previous
import jax
import jax.numpy as jnp
from jax.experimental import pallas as pl
from jax.experimental.pallas import tpu as pltpu

_THW = 25


def _fused_kernel(x_ref, pw_ref, pb_ref, w1_ref, b1_ref, w2_ref, b2_ref,
                  o_ref, h_ref):
    i = pl.program_id(0)
    thw, b, c = x_ref.shape

    pw = pw_ref[...].astype(jnp.bfloat16)
    x = x_ref[...].astype(jnp.bfloat16).reshape(thw * b, c)
    p2 = jnp.dot(x, pw, preferred_element_type=jnp.float32)
    p2t = jnp.swapaxes(p2.reshape(thw, b, 8), 0, 1).reshape(b, thw * 8)
    p2t = (p2t + pb_ref[...]).astype(jnp.bfloat16)

    w1 = w1_ref[...].reshape(thw * 8, w1_ref.shape[2])
    d = jnp.dot(p2t, w1, preferred_element_type=jnp.float32)

    @pl.when(i == 0)
    def _():
        h_ref[...] = d

    @pl.when(i > 0)
    def _():
        h_ref[...] += d

    @pl.when(i == pl.num_programs(0) - 1)
    def _():
        hr = jnp.maximum(h_ref[...] + b1_ref[...], 0.0).astype(jnp.bfloat16)
        w2 = w2_ref[...].astype(jnp.bfloat16)
        y = jnp.dot(hr, w2, preferred_element_type=jnp.float32)
        o_ref[...] = y + b2_ref[...]


@jax.jit
def _forward(feat, pool_w, pool_b, fc1_w, fc1_b, fc2_w, fc2_b):
    B, C, H, W = feat.shape
    HW = H * W
    N1 = fc1_w.shape[1]
    N2 = fc2_w.shape[1]

    xt = feat.reshape(B, C, HW).transpose(2, 0, 1)
    w1t = fc1_w.reshape(8, HW, N1).transpose(1, 0, 2).astype(jnp.bfloat16)
    pbt = jnp.tile(pool_b, _THW).reshape(1, _THW * 8)

    y = pl.pallas_call(
        _fused_kernel,
        out_shape=jax.ShapeDtypeStruct((B, N2), jnp.float32),
        grid=(HW // _THW,),
        in_specs=[
            pl.BlockSpec((_THW, B, C), lambda i: (i, 0, 0)),
            pl.BlockSpec((C, 8), lambda i: (0, 0)),
            pl.BlockSpec((1, _THW * 8), lambda i: (0, 0)),
            pl.BlockSpec((_THW, 8, N1), lambda i: (i, 0, 0)),
            pl.BlockSpec((1, N1), lambda i: (0, 0)),
            pl.BlockSpec((N1, N2), lambda i: (0, 0)),
            pl.BlockSpec((1, N2), lambda i: (0, 0)),
        ],
        out_specs=pl.BlockSpec((B, N2), lambda i: (0, 0)),
        scratch_shapes=[pltpu.VMEM((B, N1), jnp.float32)],
        compiler_params=pltpu.CompilerParams(
            dimension_semantics=("arbitrary",)),
    )(xt, pool_w.T, pbt, w1t, fc1_b.reshape(1, N1),
      fc2_w, fc2_b.reshape(1, N2))
    return y.reshape(B, 37, 10, 4)


def kernel(feat, pool_w, pool_b, fc1_w, fc1_b, fc2_w, fc2_b):
    return _forward(feat, pool_w, pool_b, fc1_w, fc1_b, fc2_w, fc2_b)


# speedup vs baseline: 2.2093x; 1.0310x over previous
import jax
import jax.numpy as jnp
from jax.experimental import pallas as pl
from jax.experimental.pallas import tpu as pltpu

_THW = 25


def _fused_kernel(x_ref, pw_ref, pb_ref, w1_ref, b1_ref, w2_ref, b2_ref,
                  o_ref, h_ref):
    i = pl.program_id(0)
    thw, b, c = x_ref.shape

    pw = pw_ref[...].astype(jnp.bfloat16)
    x = x_ref[...].astype(jnp.bfloat16).reshape(thw * b, c)
    p2 = jnp.dot(x, pw, preferred_element_type=jnp.float32)
    p2 = p2 + pb_ref[...]
    p3 = p2.reshape(thw, b, 8)

    d = None
    for ci in range(8):
        pc = jnp.swapaxes(p3[:, :, ci], 0, 1)
        pc = pc.astype(jnp.bfloat16)
        w1c = w1_ref[ci, 0].astype(jnp.bfloat16)
        dc = jnp.dot(pc, w1c, preferred_element_type=jnp.float32)
        d = dc if d is None else d + dc

    @pl.when(i == 0)
    def _():
        h_ref[...] = d

    @pl.when(i > 0)
    def _():
        h_ref[...] += d

    @pl.when(i == pl.num_programs(0) - 1)
    def _():
        hr = jnp.maximum(h_ref[...] + b1_ref[...], 0.0).astype(jnp.bfloat16)
        w2 = w2_ref[...].astype(jnp.bfloat16)
        y = jnp.dot(hr, w2, preferred_element_type=jnp.float32)
        o_ref[...] = y + b2_ref[...]


@jax.jit
def _forward(feat, pool_w, pool_b, fc1_w, fc1_b, fc2_w, fc2_b):
    B, C, H, W = feat.shape
    HW = H * W
    NT = HW // _THW
    N1 = fc1_w.shape[1]
    N2 = fc2_w.shape[1]

    xt = feat.reshape(B, C, HW).transpose(2, 0, 1)
    w1v = fc1_w.reshape(8, NT, _THW, N1)

    y = pl.pallas_call(
        _fused_kernel,
        out_shape=jax.ShapeDtypeStruct((B, N2), jnp.float32),
        grid=(NT,),
        in_specs=[
            pl.BlockSpec((_THW, B, C), lambda i: (i, 0, 0)),
            pl.BlockSpec((C, 8), lambda i: (0, 0)),
            pl.BlockSpec((1, 8), lambda i: (0, 0)),
            pl.BlockSpec((8, 1, _THW, N1), lambda i: (0, i, 0, 0)),
            pl.BlockSpec((1, N1), lambda i: (0, 0)),
            pl.BlockSpec((N1, N2), lambda i: (0, 0)),
            pl.BlockSpec((1, N2), lambda i: (0, 0)),
        ],
        out_specs=pl.BlockSpec((B, N2), lambda i: (0, 0)),
        scratch_shapes=[pltpu.VMEM((B, N1), jnp.float32)],
        compiler_params=pltpu.CompilerParams(
            dimension_semantics=("arbitrary",)),
    )(xt, pool_w.T, pool_b.reshape(1, 8), w1v, fc1_b.reshape(1, N1),
      fc2_w, fc2_b.reshape(1, N2))
    return y.reshape(B, 37, 10, 4)


def kernel(feat, pool_w, pool_b, fc1_w, fc1_b, fc2_w, fc2_b):
    return _forward(feat, pool_w, pool_b, fc1_w, fc1_b, fc2_w, fc2_b)
